# baseline (speedup 1.0000x reference)
import functools

import jax
import jax.numpy as jnp
from jax import lax
from jax.experimental import pallas as pl
from jax.experimental.pallas import tpu as pltpu
from jax.experimental.pallas import tpu_sc as plsc

N = 10000
E = 320000
D_IN = 128
D_H = 128
D_OUT = 64

NC = 2
NS = 16
EPW = E // NS
C = 125
NCHUNK = EPW // C
RPS = 640
RPS_LAST = N - (NS - 1) * RPS
ZR = 80
DW = 16

_mesh = plsc.VectorSubcoreMesh(core_axis_name="c", subcore_axis_name="s")


def _agg_body(with_deg, dc, nbuf, zr, esplit, h_hbm, src_hbm, dst_hbm,
              *rest):
    if with_deg:
        (acc_out, deg_out, srcv, dstv, rows, zbuf, zbuf_d, onesv,
         gsem, ssem, dsem, isem, acc_sh, deg_sh) = rest
    else:
        acc_out, srcv, dstv, rows, zbuf, gsem, ssem, isem, acc_sh = rest
    cid = lax.axis_index("c")
    sid = lax.axis_index("s")
    nchunk = NCHUNK // NC if esplit else NCHUNK

    if esplit:
        coff = pl.multiple_of((sid * NC + cid) * nchunk, 8)
        src_view = src_hbm.at[0]
    else:
        coff = pl.multiple_of(sid * nchunk, 8)
        src_view = src_hbm.at[cid]
    icopy_s = pltpu.make_async_copy(src_view.at[pl.ds(coff, nchunk)],
                                    srcv, isem)
    icopy_d = pltpu.make_async_copy(dst_hbm.at[1].at[pl.ds(coff, nchunk)],
                                    dstv, isem)
    icopy_s.start()
    icopy_d.start()

    nz = jnp.where(sid < NS - 1, RPS // zr, RPS_LAST // zr)
    roff = pl.multiple_of(sid * RPS, 8)

    @pl.loop(0, zr)
    def _(i):
        @pl.loop(0, dc, step=16)
        def _(j):
            zbuf[i, pl.ds(j, 16)] = jnp.zeros((16,), jnp.float32)

    @pl.loop(0, nz)
    def _(k):
        pltpu.sync_copy(zbuf, acc_sh.at[pl.ds(roff + k * zr, zr)])

    if with_deg:
        @pl.loop(0, zr)
        def _(i):
            zbuf_d[i, pl.ds(0, DW)] = jnp.zeros((DW,), jnp.float32)

        @pl.loop(0, C)
        def _(i):
            onesv[i, pl.ds(0, DW)] = jnp.ones((DW,), jnp.float32)

        @pl.loop(0, nz)
        def _(k):
            pltpu.sync_copy(zbuf_d, deg_sh.at[pl.ds(roff + k * zr, zr)])

    plsc.subcore_barrier()

    icopy_s.wait()
    icopy_d.wait()

    def gcopy(j, b):
        return pltpu.make_async_copy(h_hbm.at[srcv.at[j]], rows.at[b],
                                     gsem.at[b])

    def scopy(j, b):
        return pltpu.make_async_copy(rows.at[b], acc_sh.at[dstv.at[j]],
                                     ssem.at[b])

    for b in range(nbuf):
        gcopy(b, b).start()

    @pl.loop(0, nchunk, step=nbuf)
    def _(j):
        for b in range(nbuf):
            gcopy(j + b, b).wait()
            pltpu.async_copy(rows.at[b], acc_sh.at[dstv.at[j + b]],
                             ssem.at[b], add=True)
            if with_deg:
                @pl.when(lax.rem(j + b, 2) == cid)
                def _(b=b):
                    pltpu.async_copy(onesv, deg_sh.at[dstv.at[j + b]],
                                     dsem, add=True)
        for b in range(nbuf):
            scopy(j + b, b).wait()

            @pl.when(j + nbuf + b < nchunk)
            def _(b=b):
                gcopy(j + nbuf + b, b).start()

    if with_deg:
        @pl.loop(0, nchunk, step=2)
        def _(j):
            pltpu.make_async_copy(onesv, deg_sh.at[dstv.at[j + cid]],
                                  dsem).wait()

    plsc.subcore_barrier()

    @pl.when(sid < NS - 1)
    def _():
        pltpu.sync_copy(acc_sh.at[pl.ds(roff, RPS)],
                        acc_out.at[cid].at[pl.ds(roff, RPS)])
        if with_deg:
            pltpu.sync_copy(deg_sh.at[pl.ds(roff, RPS)],
                            deg_out.at[cid].at[pl.ds(roff, RPS)])

    @pl.when(sid == NS - 1)
    def _():
        pltpu.sync_copy(acc_sh.at[pl.ds(roff, RPS_LAST)],
                        acc_out.at[cid].at[pl.ds(roff, RPS_LAST)])
        if with_deg:
            pltpu.sync_copy(deg_sh.at[pl.ds(roff, RPS_LAST)],
                            deg_out.at[cid].at[pl.ds(roff, RPS_LAST)])


def _make_agg(with_deg, dc, nbuf, zr=ZR, esplit=False):
    nchunk = NCHUNK // NC if esplit else NCHUNK
    out_type = [jax.ShapeDtypeStruct((NC, N, dc), jnp.float32)]
    scratch = [
        pltpu.VMEM((nchunk, C), jnp.int32),
        pltpu.VMEM((nchunk, C), jnp.int32),
        pltpu.VMEM((nbuf, C, dc), jnp.float32),
        pltpu.VMEM((zr, dc), jnp.float32),
    ]
    if with_deg:
        out_type.append(jax.ShapeDtypeStruct((NC, N, DW), jnp.float32))
        scratch += [
            pltpu.VMEM((zr, DW), jnp.float32),
            pltpu.VMEM((C, DW), jnp.float32),
        ]
    scratch.append(pltpu.SemaphoreType.DMA((nbuf,)))
    scratch.append(pltpu.SemaphoreType.DMA((nbuf,)))
    if with_deg:
        scratch.append(pltpu.SemaphoreType.DMA)
    scratch.append(pltpu.SemaphoreType.DMA)
    scratch.append(pltpu.VMEM_SHARED((N, dc), jnp.float32))
    if with_deg:
        scratch.append(pltpu.VMEM_SHARED((N, DW), jnp.float32))
    return pl.kernel(
        functools.partial(_agg_body, with_deg, dc, nbuf, zr, esplit),
        out_type=tuple(out_type) if len(out_type) > 1 else out_type[0],
        mesh=_mesh,
        scratch_types=scratch,
        compiler_params=pltpu.CompilerParams(use_tc_tiling_on_sc=False),
    )


def _bn_relu(z, g, be):
    mu = jnp.mean(z, axis=0, keepdims=True)
    var = jnp.mean((z - mu) ** 2, axis=0, keepdims=True)
    z = (z - mu) * lax.rsqrt(var + 1e-5) * g + be
    return jnp.maximum(z, 0.0)


def _combine0_body(h, a, dg, ws0, wn0, b0, g0, be0, ws1, b1,
                   h1_out, zs1_out, recip_out):
    deg = dg[0, :, 0:1] + dg[1, :, 0:1]
    recip = jnp.where(deg > 0, 1.0 / jnp.maximum(deg, 1.0), 0.0)
    hn = jnp.concatenate([a[0], a[1]], axis=1) * recip
    z = jnp.dot(h[...], ws0[...], preferred_element_type=jnp.float32)
    z = z + jnp.dot(hn, wn0[...], preferred_element_type=jnp.float32) + b0[...]
    h1 = _bn_relu(jnp.maximum(z, 0.0), g0[...], be0[...])
    h1_out[...] = h1
    zs1_out[...] = jnp.dot(h1, ws1[...],
                           preferred_element_type=jnp.float32) + b1[...]
    recip_out[...] = recip


def _combine1_body(zs1, a, recip, wn1, g1, be1, wn2, ws2, b2,
                   y2_out, zs2_out):
    hn = jnp.concatenate([a[0], a[1]], axis=1) * recip[...]
    z = zs1[...] + jnp.dot(hn, wn1[...], preferred_element_type=jnp.float32)
    h2 = _bn_relu(jnp.maximum(z, 0.0), g1[...], be1[...])
    y2_out[...] = jnp.dot(h2, wn2[...], preferred_element_type=jnp.float32)
    zs2_out[...] = jnp.dot(h2, ws2[...],
                           preferred_element_type=jnp.float32) + b2[...]


def _final_body(zs, a, recip, out):
    z = zs[...] + jnp.concatenate([a[0], a[1]], axis=1) * recip[...]
    hg = jnp.mean(z, axis=0, keepdims=True)
    m = jnp.max(hg, axis=1, keepdims=True)
    e = jnp.exp(hg - m)
    out[...] = e / jnp.sum(e, axis=1, keepdims=True)


_agg_with_deg = _make_agg(True, 64, 4, 40)
_agg_plain = _make_agg(False, 64, 4)
_agg_out = _make_agg(False, 32, 4)

_f32 = jnp.float32
_combine0 = pl.pallas_call(
    _combine0_body,
    out_shape=(jax.ShapeDtypeStruct((N, D_H), _f32),
               jax.ShapeDtypeStruct((N, D_H), _f32),
               jax.ShapeDtypeStruct((N, 1), _f32)))
_combine1 = pl.pallas_call(
    _combine1_body,
    out_shape=(jax.ShapeDtypeStruct((N, D_OUT), _f32),
               jax.ShapeDtypeStruct((N, D_OUT), _f32)))
_final = pl.pallas_call(
    _final_body, out_shape=jax.ShapeDtypeStruct((1, D_OUT), _f32))


def kernel(features, edge_index, Ws0, Wn0, b0, Ws1, Wn1, b1, Ws2, Wn2, b2,
           g0, be0, g1, be1):
    src2 = edge_index[0] * 2
    srcs = jnp.stack([src2, src2 + 1]).reshape(NC, E // C, C)
    ei2 = edge_index.reshape(2, E // C, C)

    acc0, deg = _agg_with_deg(features.reshape(2 * N, 64), srcs, ei2)
    h1, zs1, recip = _combine0(features, acc0, deg, Ws0, Wn0,
                               b0.reshape(1, D_H), g0.reshape(1, D_H),
                               be0.reshape(1, D_H), Ws1, b1.reshape(1, D_H))

    acc1 = _agg_plain(h1.reshape(2 * N, 64), srcs, ei2)
    y2, zs2 = _combine1(zs1, acc1, recip, Wn1, g1.reshape(1, D_H),
                        be1.reshape(1, D_H), Wn2, Ws2,
                        b2.reshape(1, D_OUT))

    acc2 = _agg_out(y2.reshape(2 * N, 32), srcs, ei2)
    return _final(zs2, acc2, recip)


# speedup vs baseline: 1.0209x; 1.0209x over previous
import functools

import jax
import jax.numpy as jnp
from jax import lax
from jax.experimental import pallas as pl
from jax.experimental.pallas import tpu as pltpu
from jax.experimental.pallas import tpu_sc as plsc

N = 10000
E = 320000
D_IN = 128
D_H = 128
D_OUT = 64

NC = 2
NS = 16
EPW = E // NS
C = 125
NCHUNK = EPW // C
RPS = 640
RPS_LAST = N - (NS - 1) * RPS
ZR = 80
DW = 16

_mesh = plsc.VectorSubcoreMesh(core_axis_name="c", subcore_axis_name="s")


def _agg_body(with_deg, dc, nbuf, zr, esplit, h_hbm, src_hbm, dst_hbm,
              *rest):
    if with_deg:
        (acc_out, deg_out, srcv, dstv, rows, zbuf, zbuf_d, onesv,
         gsem, ssem, dsem, isem, acc_sh, deg_sh) = rest
    else:
        acc_out, srcv, dstv, rows, zbuf, gsem, ssem, isem, acc_sh = rest
    cid = lax.axis_index("c")
    sid = lax.axis_index("s")
    nchunk = NCHUNK // NC if esplit else NCHUNK

    if esplit:
        coff = pl.multiple_of((sid * NC + cid) * nchunk, 8)
        src_view = src_hbm.at[0]
    else:
        coff = pl.multiple_of(sid * nchunk, 8)
        src_view = src_hbm.at[cid]
    icopy_s = pltpu.make_async_copy(src_view.at[pl.ds(coff, nchunk)],
                                    srcv, isem)
    icopy_d = pltpu.make_async_copy(dst_hbm.at[pl.ds(coff, nchunk)],
                                    dstv, isem)
    icopy_s.start()
    icopy_d.start()

    nz = jnp.where(sid < NS - 1, RPS // zr, RPS_LAST // zr)
    roff = pl.multiple_of(sid * RPS, 8)

    @pl.loop(0, zr)
    def _(i):
        @pl.loop(0, dc, step=16)
        def _(j):
            zbuf[i, pl.ds(j, 16)] = jnp.zeros((16,), jnp.float32)

    @pl.loop(0, nz)
    def _(k):
        pltpu.sync_copy(zbuf, acc_sh.at[pl.ds(roff + k * zr, zr)])

    if with_deg:
        @pl.loop(0, zr)
        def _(i):
            zbuf_d[i, pl.ds(0, DW)] = jnp.zeros((DW,), jnp.float32)

        @pl.loop(0, C)
        def _(i):
            onesv[i, pl.ds(0, DW)] = jnp.ones((DW,), jnp.float32)

        @pl.loop(0, nz)
        def _(k):
            pltpu.sync_copy(zbuf_d, deg_sh.at[pl.ds(roff + k * zr, zr)])

    plsc.subcore_barrier()

    icopy_s.wait()
    icopy_d.wait()

    def gcopy(j, b):
        return pltpu.make_async_copy(h_hbm.at[srcv.at[j]], rows.at[b],
                                     gsem.at[b])

    def scopy(j, b):
        return pltpu.make_async_copy(rows.at[b], acc_sh.at[dstv.at[j]],
                                     ssem.at[b])

    for b in range(nbuf):
        gcopy(b, b).start()

    @pl.loop(0, nchunk, step=nbuf)
    def _(j):
        for b in range(nbuf):
            gcopy(j + b, b).wait()
            pltpu.async_copy(rows.at[b], acc_sh.at[dstv.at[j + b]],
                             ssem.at[b], add=True)
            if with_deg:
                @pl.when(lax.rem(j + b, 2) == cid)
                def _(b=b):
                    pltpu.async_copy(onesv, deg_sh.at[dstv.at[j + b]],
                                     dsem, add=True)
        for b in range(nbuf):
            scopy(j + b, b).wait()

            @pl.when(j + nbuf + b < nchunk)
            def _(b=b):
                gcopy(j + nbuf + b, b).start()

    if with_deg:
        @pl.loop(0, nchunk, step=2)
        def _(j):
            pltpu.make_async_copy(onesv, deg_sh.at[dstv.at[j + cid]],
                                  dsem).wait()

    plsc.subcore_barrier()

    @pl.when(sid < NS - 1)
    def _():
        pltpu.sync_copy(acc_sh.at[pl.ds(roff, RPS)],
                        acc_out.at[cid].at[pl.ds(roff, RPS)])
        if with_deg:
            pltpu.sync_copy(deg_sh.at[pl.ds(roff, RPS)],
                            deg_out.at[cid].at[pl.ds(roff, RPS)])

    @pl.when(sid == NS - 1)
    def _():
        pltpu.sync_copy(acc_sh.at[pl.ds(roff, RPS_LAST)],
                        acc_out.at[cid].at[pl.ds(roff, RPS_LAST)])
        if with_deg:
            pltpu.sync_copy(deg_sh.at[pl.ds(roff, RPS_LAST)],
                            deg_out.at[cid].at[pl.ds(roff, RPS_LAST)])


def _make_agg(with_deg, dc, nbuf, zr=ZR, esplit=False):
    nchunk = NCHUNK // NC if esplit else NCHUNK
    out_type = [jax.ShapeDtypeStruct((NC, N, dc), jnp.float32)]
    scratch = [
        pltpu.VMEM((nchunk, C), jnp.int32),
        pltpu.VMEM((nchunk, C), jnp.int32),
        pltpu.VMEM((nbuf, C, dc), jnp.float32),
        pltpu.VMEM((zr, dc), jnp.float32),
    ]
    if with_deg:
        out_type.append(jax.ShapeDtypeStruct((NC, N, DW), jnp.float32))
        scratch += [
            pltpu.VMEM((zr, DW), jnp.float32),
            pltpu.VMEM((C, DW), jnp.float32),
        ]
    scratch.append(pltpu.SemaphoreType.DMA((nbuf,)))
    scratch.append(pltpu.SemaphoreType.DMA((nbuf,)))
    if with_deg:
        scratch.append(pltpu.SemaphoreType.DMA)
    scratch.append(pltpu.SemaphoreType.DMA)
    scratch.append(pltpu.VMEM_SHARED((N, dc), jnp.float32))
    if with_deg:
        scratch.append(pltpu.VMEM_SHARED((N, DW), jnp.float32))
    return pl.kernel(
        functools.partial(_agg_body, with_deg, dc, nbuf, zr, esplit),
        out_type=tuple(out_type) if len(out_type) > 1 else out_type[0],
        mesh=_mesh,
        scratch_types=scratch,
        compiler_params=pltpu.CompilerParams(use_tc_tiling_on_sc=False),
    )


def _bn_relu(z, g, be):
    mu = jnp.mean(z, axis=0, keepdims=True)
    var = jnp.mean((z - mu) ** 2, axis=0, keepdims=True)
    z = (z - mu) * lax.rsqrt(var + 1e-5) * g + be
    return jnp.maximum(z, 0.0)


def _combine0_body(h, a, dg, ws0, wn0, b0, g0, be0, ws1, b1,
                   h1_out, zs1_out, recip_out):
    deg = dg[0, :, 0:1] + dg[1, :, 0:1]
    recip = jnp.where(deg > 0, 1.0 / jnp.maximum(deg, 1.0), 0.0)
    hn = jnp.concatenate([a[0], a[1]], axis=1) * recip
    z = jnp.dot(h[...], ws0[...], preferred_element_type=jnp.float32)
    z = z + jnp.dot(hn, wn0[...], preferred_element_type=jnp.float32) + b0[...]
    h1 = _bn_relu(jnp.maximum(z, 0.0), g0[...], be0[...])
    h1_out[...] = h1
    zs1_out[...] = jnp.dot(h1, ws1[...],
                           preferred_element_type=jnp.float32) + b1[...]
    recip_out[...] = recip


def _combine1_body(zs1, a, recip, wn1, g1, be1, wn2, ws2, b2,
                   y2_out, zs2_out):
    hn = jnp.concatenate([a[0], a[1]], axis=1) * recip[...]
    z = zs1[...] + jnp.dot(hn, wn1[...], preferred_element_type=jnp.float32)
    h2 = _bn_relu(jnp.maximum(z, 0.0), g1[...], be1[...])
    y2_out[...] = jnp.dot(h2, wn2[...], preferred_element_type=jnp.float32)
    zs2_out[...] = jnp.dot(h2, ws2[...],
                           preferred_element_type=jnp.float32) + b2[...]


def _final_body(zs, a, recip, out):
    z = zs[...] + jnp.concatenate([a[0], a[1]], axis=1) * recip[...]
    hg = jnp.mean(z, axis=0, keepdims=True)
    m = jnp.max(hg, axis=1, keepdims=True)
    e = jnp.exp(hg - m)
    out[...] = e / jnp.sum(e, axis=1, keepdims=True)


_agg_with_deg = _make_agg(True, 64, 4, 40)
_agg_plain = _make_agg(False, 64, 4)
_agg_out = _make_agg(False, 32, 4)

_f32 = jnp.float32
_combine0 = pl.pallas_call(
    _combine0_body,
    out_shape=(jax.ShapeDtypeStruct((N, D_H), _f32),
               jax.ShapeDtypeStruct((N, D_H), _f32),
               jax.ShapeDtypeStruct((N, 1), _f32)))
_combine1 = pl.pallas_call(
    _combine1_body,
    out_shape=(jax.ShapeDtypeStruct((N, D_OUT), _f32),
               jax.ShapeDtypeStruct((N, D_OUT), _f32)))
_final = pl.pallas_call(
    _final_body, out_shape=jax.ShapeDtypeStruct((1, D_OUT), _f32))


def kernel(features, edge_index, Ws0, Wn0, b0, Ws1, Wn1, b1, Ws2, Wn2, b2,
           g0, be0, g1, be1):
    src2 = edge_index[0] * 2
    srcs = jnp.stack([src2, src2 + 1]).reshape(NC, E // C, C)
    dst2d = edge_index[1].reshape(E // C, C)

    acc0, deg = _agg_with_deg(features.reshape(2 * N, 64), srcs, dst2d)
    h1, zs1, recip = _combine0(features, acc0, deg, Ws0, Wn0,
                               b0.reshape(1, D_H), g0.reshape(1, D_H),
                               be0.reshape(1, D_H), Ws1, b1.reshape(1, D_H))

    acc1 = _agg_plain(h1.reshape(2 * N, 64), srcs, dst2d)
    y2, zs2 = _combine1(zs1, acc1, recip, Wn1, g1.reshape(1, D_H),
                        be1.reshape(1, D_H), Wn2, Ws2,
                        b2.reshape(1, D_OUT))

    acc2 = _agg_out(y2.reshape(2 * N, 32), srcs, dst2d)
    return _final(zs2, acc2, recip)
